# baseline (speedup 1.0000x reference)
import functools

import jax
import jax.numpy as jnp
from jax import lax
from jax.experimental import pallas as pl
from jax.experimental.pallas import tpu as pltpu
from jax.experimental.pallas import tpu_sc as plsc

N = 10000
NPAD = 10240
E = 320000
F = 128
NC = 2
NS = 16
NW = NC * NS
C = 80
NCH = 132
EPT = NCH * C
E2 = EPT * NW
RPT = NPAD // NS
ZR = 128

_mesh = plsc.VectorSubcoreMesh(core_axis_name="c", subcore_axis_name="s")


@functools.partial(
    pl.kernel,
    out_type=jax.ShapeDtypeStruct((NC, NPAD), jnp.float32),
    mesh=_mesh,
    scratch_types=[
        pltpu.VMEM_SHARED((NPAD,), jnp.float32),
        pltpu.VMEM((EPT,), jnp.int32),
        pltpu.VMEM((EPT,), jnp.float32),
        pltpu.VMEM((C,), jnp.int32),
        pltpu.VMEM((C,), jnp.float32),
        pltpu.VMEM((RPT,), jnp.float32),
    ],
)
def _deg_kernel(dst_hbm, ew_hbm, out_hbm, acc, dstall, ewall, dstbuf, ewbuf, zb):
    cid = lax.axis_index("c")
    sid = lax.axis_index("s")
    wid = cid * NS + sid
    eb = wid * EPT
    pltpu.sync_copy(dst_hbm.at[pl.ds(eb, EPT)], dstall)
    pltpu.sync_copy(ew_hbm.at[pl.ds(eb, EPT)], ewall)

    def zb_body(i, carry):
        zb[pl.ds(i * 16, 16)] = jnp.zeros((16,), jnp.float32)
        return carry

    lax.fori_loop(0, RPT // 16, zb_body, 0)
    pltpu.sync_copy(zb, acc.at[pl.ds(sid * RPT, RPT)])
    plsc.subcore_barrier()

    def ch_body(c, carry):
        off = c * C
        for g in range(C // 16):
            dstbuf[pl.ds(g * 16, 16)] = dstall[pl.ds(off + g * 16, 16)]
            ewbuf[pl.ds(g * 16, 16)] = ewall[pl.ds(off + g * 16, 16)]
        pltpu.sync_copy(ewbuf, acc.at[dstbuf], add=True)
        return carry

    lax.fori_loop(0, NCH, ch_body, 0)
    plsc.subcore_barrier()
    pltpu.sync_copy(acc.at[pl.ds(sid * RPT, RPT)],
                    out_hbm.at[cid, pl.ds(sid * RPT, RPT)])


def _mid_body(x_ref, w_ref, degp_ref, hs_ref, dis_ref):
    hs_ref[...] = jnp.dot(x_ref[...], w_ref[...],
                          preferred_element_type=jnp.float32)
    deg = degp_ref[0, :] + degp_ref[1, :]
    dis_ref[...] = jnp.where(deg > 0, lax.rsqrt(deg), 0.0)[None, :]


_RB = 1024

_tc_mid = pl.pallas_call(
    _mid_body,
    grid=(NPAD // _RB,),
    in_specs=[
        pl.BlockSpec((_RB, F), lambda i: (i, 0)),
        pl.BlockSpec((F, F), lambda i: (0, 0)),
        pl.BlockSpec((NC, _RB), lambda i: (0, i)),
    ],
    out_specs=[
        pl.BlockSpec((_RB, F), lambda i: (i, 0)),
        pl.BlockSpec((1, _RB), lambda i: (0, i)),
    ],
    out_shape=[
        jax.ShapeDtypeStruct((NPAD, F), jnp.float32),
        jax.ShapeDtypeStruct((1, NPAD), jnp.float32),
    ],
)


_G = C // 16
SUP = 6
SUPC = SUP * C
BODY = 2 * SUP
NBODY = NCH // BODY


@functools.partial(
    pl.kernel,
    out_type=jax.ShapeDtypeStruct((NC, NPAD, F), jnp.float32),
    mesh=_mesh,
    scratch_types=[
        pltpu.VMEM_SHARED((NPAD, F), jnp.float32),
        pltpu.VMEM((NPAD,), jnp.float32),
        pltpu.VMEM((C,), jnp.float32),
        pltpu.VMEM((SUPC,), jnp.int32),
        pltpu.VMEM((SUPC,), jnp.int32),
        pltpu.VMEM((SUPC,), jnp.int32),
        pltpu.VMEM((SUPC,), jnp.int32),
        pltpu.VMEM((SUPC,), jnp.float32),
        pltpu.VMEM((SUPC,), jnp.float32),
    ] + [pltpu.VMEM((C, F), jnp.float32)] * 3
      + [pltpu.VMEM((C,), jnp.int32)] * 3
      + [pltpu.SemaphoreType.DMA] * 8,
    compiler_params=pltpu.CompilerParams(needs_layout_passes=False),
)
def _msg_kernel(hs_hbm, src_hbm, dst_hbm, ew_hbm, dis_hbm, out_hbm,
                acc, disv, sbuf, srcSupA, srcSupB, dstSupA, dstSupB,
                ewSupA, ewSupB, rows0, rows1, rows2, dS0, dS1, dS2,
                sem0, sem1, sem2, semS0, semS1, semS2, semEA, semEB):
    cid = lax.axis_index("c")
    sid = lax.axis_index("s")
    wid = cid * NS + sid
    eb = wid * EPT
    pltpu.sync_copy(dis_hbm.at[0], disv)

    def zb_body(i, carry):
        for j in range(F // 16):
            rows0[i, pl.ds(j * 16, 16)] = jnp.zeros((16,), jnp.float32)
        return carry

    lax.fori_loop(0, C, zb_body, 0)
    for k in range(RPT // C):
        pltpu.sync_copy(rows0, acc.at[pl.ds(sid * RPT + k * C, C)])
    plsc.subcore_barrier()

    supA = (srcSupA, dstSupA, ewSupA, semEA)
    supB = (srcSupB, dstSupB, ewSupB, semEB)
    rot = ((rows0, dS0, sem0, semS0),
           (rows1, dS1, sem1, semS1),
           (rows2, dS2, sem2, semS2))

    def sup_load(sup, start):
        srcs, dsts, ews, semE = sup
        off = eb + start * C
        pltpu.async_copy(src_hbm.at[pl.ds(off, SUPC)], srcs, semE)
        pltpu.async_copy(dst_hbm.at[pl.ds(off, SUPC)], dsts, semE)
        pltpu.async_copy(ew_hbm.at[pl.ds(off, SUPC)], ews, semE)

    def sup_wait(sup, start):
        srcs, dsts, ews, semE = sup
        off = eb + start * C
        pltpu.make_async_copy(src_hbm.at[pl.ds(off, SUPC)], srcs, semE).wait()
        pltpu.make_async_copy(dst_hbm.at[pl.ds(off, SUPC)], dsts, semE).wait()
        pltpu.make_async_copy(ew_hbm.at[pl.ds(off, SUPC)], ews, semE).wait()

    def gather(sup, koff, rows, sem):
        pltpu.async_copy(hs_hbm.at[sup[0].at[pl.ds(koff * C, C)]], rows, sem)

    def gather_wait(sup, koff, rows, sem):
        pltpu.make_async_copy(hs_hbm.at[sup[0].at[pl.ds(koff * C, C)]],
                              rows, sem).wait()

    def process(sup, koff, rows, dstS, semS):
        srcs, dsts, ews, _ = sup
        be = koff * C
        for g in range(_G):
            sl = pl.ds(g * 16, 16)
            esl = pl.ds(be + g * 16, 16)
            d16 = dsts[esl]
            dstS[sl] = d16
            sbuf[sl] = (ews[esl] * plsc.load_gather(disv, [srcs[esl]])
                        * plsc.load_gather(disv, [d16]))

        def gbody(g, carry):
            sv = sbuf[pl.ds(g * 16, 16)]
            base = g * 16
            for k in range(16):
                s = sv[k]
                for j in range(F // 16):
                    rows[base + k, pl.ds(j * 16, 16)] = (
                        rows[base + k, pl.ds(j * 16, 16)] * s)
            return carry

        lax.fori_loop(0, _G, gbody, 0)
        pltpu.async_copy(rows, acc.at[dstS], semS, add=True)

    def scatter_wait(rows, dstS, semS):
        pltpu.make_async_copy(rows, acc.at[dstS], semS).wait()

    sup_load(supA, 0)
    sup_load(supB, SUP)
    sup_wait(supA, 0)
    gather(supA, 0, rows0, sem0)
    gather(supA, 1, rows1, sem1)

    def body(i, carry):
        base = i * BODY
        for k in range(BODY):
            cur_rows, cur_dS, cur_sem, cur_semS = rot[k % 3]
            nxt_rows, nxt_dS, nxt_sem, nxt_semS = rot[(k + 2) % 3]
            supc = supA if k < SUP else supB
            offc = k if k < SUP else k - SUP
            c = base + k

            if k + 2 < BODY:
                supn = supA if k + 2 < SUP else supB
                offn = k + 2 if k + 2 < SUP else k + 2 - SUP
                if k == 4:
                    sup_wait(supB, base + SUP)
                if k == 0:
                    @pl.when(c >= 1)
                    def _():
                        scatter_wait(nxt_rows, nxt_dS, nxt_semS)
                else:
                    scatter_wait(nxt_rows, nxt_dS, nxt_semS)
                gather(supn, offn, nxt_rows, nxt_sem)
            else:
                @pl.when(c + 2 < NCH)
                def _(k=k, c=c, nxt_rows=nxt_rows, nxt_dS=nxt_dS,
                      nxt_sem=nxt_sem, nxt_semS=nxt_semS):
                    if k == 10:
                        sup_wait(supA, base + BODY)
                    scatter_wait(nxt_rows, nxt_dS, nxt_semS)
                    gather(supA, k - 10, nxt_rows, nxt_sem)

            gather_wait(supc, offc, cur_rows, cur_sem)
            process(supc, offc, cur_rows, cur_dS, cur_semS)

            if k == SUP:
                @pl.when(i < NBODY - 1)
                def _():
                    sup_load(supA, base + BODY)
            if k == BODY - 1:
                @pl.when(i < NBODY - 1)
                def _():
                    sup_load(supB, base + BODY + SUP)
        return carry

    lax.fori_loop(0, NBODY, body, 0)

    scatter_wait(rows0, dS0, semS0)
    scatter_wait(rows1, dS1, semS1)
    scatter_wait(rows2, dS2, semS2)
    plsc.subcore_barrier()
    for k in range(RPT // C):
        r0 = sid * RPT + k * C
        pltpu.sync_copy(acc.at[pl.ds(r0, C)],
                        out_hbm.at[cid, pl.ds(r0, C)])


def _post_body(accp_ref, b_ref, g_ref, be_ref, rm_ref, rv_ref, lg_ref, lb_ref,
               o_ref):
    o = accp_ref[0] + accp_ref[1] + b_ref[...]
    o = jnp.where(o > 0, o, jnp.exp(o) - 1.0)
    o = (o - rm_ref[...]) * lax.rsqrt(rv_ref[...] + 1e-5) * g_ref[...] + be_ref[...]
    mu = jnp.mean(o, axis=-1, keepdims=True)
    var = jnp.mean((o - mu) ** 2, axis=-1, keepdims=True)
    o_ref[...] = (o - mu) * lax.rsqrt(var + 1e-5) * lg_ref[...] + lb_ref[...]


_vec_spec = pl.BlockSpec((1, F), lambda i: (0, 0))

_tc_post = pl.pallas_call(
    _post_body,
    grid=(NPAD // _RB,),
    in_specs=[pl.BlockSpec((NC, _RB, F), lambda i: (0, i, 0))] + [_vec_spec] * 7,
    out_specs=pl.BlockSpec((_RB, F), lambda i: (i, 0)),
    out_shape=jax.ShapeDtypeStruct((NPAD, F), jnp.float32),
)


def kernel(x, edge_index, edge_weight, W, b, bn_gamma, bn_beta,
           running_mean, running_var, ln_gamma, ln_beta):
    loop = jnp.arange(NPAD, dtype=jnp.int32)
    npadedge = E2 - E - NPAD
    src2 = jnp.concatenate(
        [edge_index[0], loop, jnp.zeros((npadedge,), jnp.int32)])
    dst2 = jnp.concatenate(
        [edge_index[1], loop, jnp.zeros((npadedge,), jnp.int32)])
    ew2 = jnp.concatenate(
        [edge_weight, jnp.ones((NPAD,), jnp.float32),
         jnp.zeros((npadedge,), jnp.float32)])
    xpad = jnp.pad(x, ((0, NPAD - N), (0, 0)))

    degp = _deg_kernel(dst2, ew2)
    hs, dis = _tc_mid(xpad, W, degp)
    accp = _msg_kernel(hs, src2, dst2, ew2, dis)
    out = _tc_post(accp, b.reshape(1, F), bn_gamma.reshape(1, F),
                   bn_beta.reshape(1, F), running_mean.reshape(1, F),
                   running_var.reshape(1, F), ln_gamma.reshape(1, F),
                   ln_beta.reshape(1, F))
    return out[:N]


# speedup vs baseline: 1.9764x; 1.9764x over previous
import functools

import jax
import jax.numpy as jnp
from jax import lax
from jax.experimental import pallas as pl
from jax.experimental.pallas import tpu as pltpu
from jax.experimental.pallas import tpu_sc as plsc

N = 10000
NPAD = 10240
E = 320000
F = 128
NC = 2
NS = 16
NW = NC * NS
C = 64
NCH = 162
EPT = NCH * C
E2 = EPT * NW
RPT = NPAD // NS
ZR = 128

_mesh = plsc.VectorSubcoreMesh(core_axis_name="c", subcore_axis_name="s")


@functools.partial(
    pl.kernel,
    out_type=jax.ShapeDtypeStruct((NC, NPAD), jnp.float32),
    mesh=_mesh,
    scratch_types=[
        pltpu.VMEM_SHARED((NPAD,), jnp.float32),
        pltpu.VMEM((EPT,), jnp.int32),
        pltpu.VMEM((EPT,), jnp.float32),
        pltpu.VMEM((C,), jnp.int32),
        pltpu.VMEM((C,), jnp.float32),
        pltpu.VMEM((RPT,), jnp.float32),
    ],
)
def _deg_kernel(dst_hbm, ew_hbm, out_hbm, acc, dstall, ewall, dstbuf, ewbuf, zb):
    cid = lax.axis_index("c")
    sid = lax.axis_index("s")
    wid = cid * NS + sid
    eb = wid * EPT
    pltpu.sync_copy(dst_hbm.at[pl.ds(eb, EPT)], dstall)
    pltpu.sync_copy(ew_hbm.at[pl.ds(eb, EPT)], ewall)

    def zb_body(i, carry):
        zb[pl.ds(i * 16, 16)] = jnp.zeros((16,), jnp.float32)
        return carry

    lax.fori_loop(0, RPT // 16, zb_body, 0)
    pltpu.sync_copy(zb, acc.at[pl.ds(sid * RPT, RPT)])
    plsc.subcore_barrier()

    def ch_body(c, carry):
        off = c * C
        for g in range(C // 16):
            dstbuf[pl.ds(g * 16, 16)] = dstall[pl.ds(off + g * 16, 16)]
            ewbuf[pl.ds(g * 16, 16)] = ewall[pl.ds(off + g * 16, 16)]
        pltpu.sync_copy(ewbuf, acc.at[dstbuf], add=True)
        return carry

    lax.fori_loop(0, NCH, ch_body, 0)
    plsc.subcore_barrier()
    pltpu.sync_copy(acc.at[pl.ds(sid * RPT, RPT)],
                    out_hbm.at[cid, pl.ds(sid * RPT, RPT)])


def _mid_body(x_ref, w_ref, degp_ref, hs_ref, dis_ref):
    hs_ref[...] = jnp.dot(x_ref[...], w_ref[...],
                          preferred_element_type=jnp.float32)
    deg = degp_ref[0, :] + degp_ref[1, :]
    dis_ref[...] = jnp.where(deg > 0, lax.rsqrt(deg), 0.0)[None, :]


_RB = 1024

_tc_mid = pl.pallas_call(
    _mid_body,
    grid=(NPAD // _RB,),
    in_specs=[
        pl.BlockSpec((_RB, F), lambda i: (i, 0)),
        pl.BlockSpec((F, F), lambda i: (0, 0)),
        pl.BlockSpec((NC, _RB), lambda i: (0, i)),
    ],
    out_specs=[
        pl.BlockSpec((_RB, F), lambda i: (i, 0)),
        pl.BlockSpec((1, _RB), lambda i: (0, i)),
    ],
    out_shape=[
        jax.ShapeDtypeStruct((NPAD, F), jnp.float32),
        jax.ShapeDtypeStruct((1, NPAD), jnp.float32),
    ],
)


_G = C // 16


@functools.partial(
    pl.kernel,
    out_type=jax.ShapeDtypeStruct((NC, NPAD, F), jnp.float32),
    mesh=_mesh,
    scratch_types=[
        pltpu.VMEM_SHARED((NPAD, F), jnp.float32),
        pltpu.VMEM((NPAD,), jnp.float32),
        pltpu.VMEM((C,), jnp.float32),
    ] + [pltpu.VMEM((C,), jnp.int32)] * 3
      + [pltpu.VMEM((C,), jnp.int32)] * 3
      + [pltpu.VMEM((C,), jnp.float32)] * 3
      + [pltpu.VMEM((C, F), jnp.float32)] * 3
      + [pltpu.VMEM((C,), jnp.int32)] * 3
      + [pltpu.SemaphoreType.DMA] * 6,
    compiler_params=pltpu.CompilerParams(needs_layout_passes=False),
)
def _msg_kernel(hs_hbm, src_hbm, dst_hbm, ew_hbm, dis_hbm, out_hbm,
                acc, disv, sbuf, src0, src1, src2, dst0, dst1, dst2,
                ew0, ew1, ew2, rows0, rows1, rows2, dS0, dS1, dS2,
                sem0, sem1, sem2, semS0, semS1, semS2):
    cid = lax.axis_index("c")
    sid = lax.axis_index("s")
    wid = cid * NS + sid
    eb = wid * EPT
    pltpu.sync_copy(dis_hbm.at[0], disv)

    def zb_body(i, carry):
        for j in range(F // 16):
            rows0[i, pl.ds(j * 16, 16)] = jnp.zeros((16,), jnp.float32)
        return carry

    lax.fori_loop(0, C, zb_body, 0)
    for k in range(RPT // C):
        pltpu.sync_copy(rows0, acc.at[pl.ds(sid * RPT + k * C, C)])
    plsc.subcore_barrier()

    def e_load(c, srcb, dstb, ewb, sem):
        off = eb + c * C
        pltpu.async_copy(src_hbm.at[pl.ds(off, C)], srcb, sem)
        pltpu.async_copy(dst_hbm.at[pl.ds(off, C)], dstb, sem)
        pltpu.async_copy(ew_hbm.at[pl.ds(off, C)], ewb, sem)

    def e_wait(c, srcb, dstb, ewb, sem):
        off = eb + c * C
        pltpu.make_async_copy(src_hbm.at[pl.ds(off, C)], srcb, sem).wait()
        pltpu.make_async_copy(dst_hbm.at[pl.ds(off, C)], dstb, sem).wait()
        pltpu.make_async_copy(ew_hbm.at[pl.ds(off, C)], ewb, sem).wait()

    def gather(srcb, rows, sem):
        pltpu.async_copy(hs_hbm.at[srcb], rows, sem)

    def gather_wait(srcb, rows, sem):
        pltpu.make_async_copy(hs_hbm.at[srcb], rows, sem).wait()

    def process(srcb, dstb, ewb, rows, dstS, semS):
        for g in range(_G):
            sl = pl.ds(g * 16, 16)
            d16 = dstb[sl]
            dstS[sl] = d16
            sbuf[sl] = (ewb[sl] * plsc.load_gather(disv, [srcb[sl]])
                        * plsc.load_gather(disv, [d16]))

        def gbody(g, carry):
            sv = sbuf[pl.ds(g * 16, 16)]
            base = g * 16
            for k in range(16):
                s = sv[k]
                for j in range(F // 16):
                    rows[base + k, pl.ds(j * 16, 16)] = (
                        rows[base + k, pl.ds(j * 16, 16)] * s)
            return carry

        lax.fori_loop(0, _G, gbody, 0)
        pltpu.async_copy(rows, acc.at[dstS], semS, add=True)

    def scatter_wait(rows, dstS, semS):
        pltpu.make_async_copy(rows, acc.at[dstS], semS).wait()

    buf0 = (src0, dst0, ew0, rows0, dS0, sem0, semS0)
    buf1 = (src1, dst1, ew1, rows1, dS1, sem1, semS1)
    buf2 = (src2, dst2, ew2, rows2, dS2, sem2, semS2)

    e_load(0, src0, dst0, ew0, sem0)
    e_load(1, src1, dst1, ew1, sem1)
    e_load(2, src2, dst2, ew2, sem2)
    e_wait(0, src0, dst0, ew0, sem0)
    gather(src0, rows0, sem0)
    e_wait(1, src1, dst1, ew1, sem1)
    gather(src1, rows1, sem1)

    def step(i, cur, nxt2):
        srcb, dstb, ewb, rows, dstS, sem, semS = cur
        srcN, dstN, ewN, rowsN, dstSN, semN, semSN = nxt2

        @pl.when(i < NCH - 2)
        def _():
            e_wait(i + 2, srcN, dstN, ewN, semN)

            @pl.when(i >= 1)
            def _():
                scatter_wait(rowsN, dstSN, semSN)

            gather(srcN, rowsN, semN)

        gather_wait(srcb, rows, sem)
        process(srcb, dstb, ewb, rows, dstS, semS)

        @pl.when(i < NCH - 3)
        def _():
            e_load(i + 3, srcb, dstb, ewb, sem)

    def body(i, carry):
        step(3 * i, buf0, buf2)
        step(3 * i + 1, buf1, buf0)
        step(3 * i + 2, buf2, buf1)
        return carry

    lax.fori_loop(0, NCH // 3, body, 0)

    scatter_wait(rows0, dS0, semS0)
    scatter_wait(rows1, dS1, semS1)
    scatter_wait(rows2, dS2, semS2)
    plsc.subcore_barrier()
    for k in range(RPT // C):
        r0 = sid * RPT + k * C
        pltpu.sync_copy(acc.at[pl.ds(r0, C)],
                        out_hbm.at[cid, pl.ds(r0, C)])


def _post_body(accp_ref, b_ref, g_ref, be_ref, rm_ref, rv_ref, lg_ref, lb_ref,
               o_ref):
    o = accp_ref[0] + accp_ref[1] + b_ref[...]
    o = jnp.where(o > 0, o, jnp.exp(o) - 1.0)
    o = (o - rm_ref[...]) * lax.rsqrt(rv_ref[...] + 1e-5) * g_ref[...] + be_ref[...]
    mu = jnp.mean(o, axis=-1, keepdims=True)
    var = jnp.mean((o - mu) ** 2, axis=-1, keepdims=True)
    o_ref[...] = (o - mu) * lax.rsqrt(var + 1e-5) * lg_ref[...] + lb_ref[...]


_vec_spec = pl.BlockSpec((1, F), lambda i: (0, 0))

_tc_post = pl.pallas_call(
    _post_body,
    grid=(NPAD // _RB,),
    in_specs=[pl.BlockSpec((NC, _RB, F), lambda i: (0, i, 0))] + [_vec_spec] * 7,
    out_specs=pl.BlockSpec((_RB, F), lambda i: (i, 0)),
    out_shape=jax.ShapeDtypeStruct((NPAD, F), jnp.float32),
)


def kernel(x, edge_index, edge_weight, W, b, bn_gamma, bn_beta,
           running_mean, running_var, ln_gamma, ln_beta):
    loop = jnp.arange(NPAD, dtype=jnp.int32)
    npadedge = E2 - E - NPAD
    src2 = jnp.concatenate(
        [edge_index[0], loop, jnp.zeros((npadedge,), jnp.int32)])
    dst2 = jnp.concatenate(
        [edge_index[1], loop, jnp.zeros((npadedge,), jnp.int32)])
    ew2 = jnp.concatenate(
        [edge_weight, jnp.ones((NPAD,), jnp.float32),
         jnp.zeros((npadedge,), jnp.float32)])
    xpad = jnp.pad(x, ((0, NPAD - N), (0, 0)))

    degp = _deg_kernel(dst2, ew2)
    hs, dis = _tc_mid(xpad, W, degp)
    accp = _msg_kernel(hs, src2, dst2, ew2, dis)
    out = _tc_post(accp, b.reshape(1, F), bn_gamma.reshape(1, F),
                   bn_beta.reshape(1, F), running_mean.reshape(1, F),
                   running_var.reshape(1, F), ln_gamma.reshape(1, F),
                   ln_beta.reshape(1, F))
    return out[:N]


# speedup vs baseline: 2.5137x; 1.2719x over previous
import functools

import jax
import jax.numpy as jnp
from jax import lax
from jax.experimental import pallas as pl
from jax.experimental.pallas import tpu as pltpu
from jax.experimental.pallas import tpu_sc as plsc

N = 10000
NPAD = 10240
E = 320000
F = 128
NC = 2
NS = 16
NW = NC * NS
C = 80
NCH = 129
EPT = NCH * C
E2 = EPT * NW
RPT = NPAD // NS
ZR = 128

_mesh = plsc.VectorSubcoreMesh(core_axis_name="c", subcore_axis_name="s")


@functools.partial(
    pl.kernel,
    out_type=jax.ShapeDtypeStruct((NC, NPAD), jnp.float32),
    mesh=_mesh,
    scratch_types=[
        pltpu.VMEM_SHARED((NPAD,), jnp.float32),
        pltpu.VMEM((EPT,), jnp.int32),
        pltpu.VMEM((EPT,), jnp.float32),
        pltpu.VMEM((C,), jnp.int32),
        pltpu.VMEM((C,), jnp.float32),
        pltpu.VMEM((RPT,), jnp.float32),
    ],
)
def _deg_kernel(dst_hbm, ew_hbm, out_hbm, acc, dstall, ewall, dstbuf, ewbuf, zb):
    cid = lax.axis_index("c")
    sid = lax.axis_index("s")
    wid = cid * NS + sid
    eb = wid * EPT
    pltpu.sync_copy(dst_hbm.at[pl.ds(eb, EPT)], dstall)
    pltpu.sync_copy(ew_hbm.at[pl.ds(eb, EPT)], ewall)

    def zb_body(i, carry):
        zb[pl.ds(i * 16, 16)] = jnp.zeros((16,), jnp.float32)
        return carry

    lax.fori_loop(0, RPT // 16, zb_body, 0)
    pltpu.sync_copy(zb, acc.at[pl.ds(sid * RPT, RPT)])
    plsc.subcore_barrier()

    def ch_body(c, carry):
        off = c * C
        for g in range(C // 16):
            dstbuf[pl.ds(g * 16, 16)] = dstall[pl.ds(off + g * 16, 16)]
            ewbuf[pl.ds(g * 16, 16)] = ewall[pl.ds(off + g * 16, 16)]
        pltpu.sync_copy(ewbuf, acc.at[dstbuf], add=True)
        return carry

    lax.fori_loop(0, NCH, ch_body, 0)
    plsc.subcore_barrier()
    pltpu.sync_copy(acc.at[pl.ds(sid * RPT, RPT)],
                    out_hbm.at[cid, pl.ds(sid * RPT, RPT)])


def _mid_body(x_ref, w_ref, degp_ref, hs_ref, dis_ref):
    hs_ref[...] = jnp.dot(x_ref[...], w_ref[...],
                          preferred_element_type=jnp.float32)
    deg = degp_ref[0, :] + degp_ref[1, :]
    dis_ref[...] = jnp.where(deg > 0, lax.rsqrt(deg), 0.0)[None, :]


_RB = 1024

_tc_mid = pl.pallas_call(
    _mid_body,
    grid=(NPAD // _RB,),
    in_specs=[
        pl.BlockSpec((_RB, F), lambda i: (i, 0)),
        pl.BlockSpec((F, F), lambda i: (0, 0)),
        pl.BlockSpec((NC, _RB), lambda i: (0, i)),
    ],
    out_specs=[
        pl.BlockSpec((_RB, F), lambda i: (i, 0)),
        pl.BlockSpec((1, _RB), lambda i: (0, i)),
    ],
    out_shape=[
        jax.ShapeDtypeStruct((NPAD, F), jnp.float32),
        jax.ShapeDtypeStruct((1, NPAD), jnp.float32),
    ],
)


_G = C // 16


@functools.partial(
    pl.kernel,
    out_type=jax.ShapeDtypeStruct((NC, NPAD, F), jnp.float32),
    mesh=_mesh,
    scratch_types=[
        pltpu.VMEM_SHARED((NPAD, F), jnp.float32),
        pltpu.VMEM((NPAD,), jnp.float32),
        pltpu.VMEM((C,), jnp.float32),
    ] + [pltpu.VMEM((C,), jnp.int32)] * 3
      + [pltpu.VMEM((C,), jnp.int32)] * 3
      + [pltpu.VMEM((C,), jnp.float32)] * 3
      + [pltpu.VMEM((C, F), jnp.float32)] * 3
      + [pltpu.VMEM((C,), jnp.int32)] * 3
      + [pltpu.SemaphoreType.DMA] * 6,
    compiler_params=pltpu.CompilerParams(needs_layout_passes=False),
)
def _msg_kernel(hs_hbm, src_hbm, dst_hbm, ew_hbm, dis_hbm, out_hbm,
                acc, disv, sbuf, src0, src1, src2, dst0, dst1, dst2,
                ew0, ew1, ew2, rows0, rows1, rows2, dS0, dS1, dS2,
                sem0, sem1, sem2, semS0, semS1, semS2):
    cid = lax.axis_index("c")
    sid = lax.axis_index("s")
    wid = cid * NS + sid
    eb = wid * EPT
    pltpu.sync_copy(dis_hbm.at[0], disv)

    def zb_body(i, carry):
        for j in range(F // 16):
            rows0[i, pl.ds(j * 16, 16)] = jnp.zeros((16,), jnp.float32)
        return carry

    lax.fori_loop(0, C, zb_body, 0)
    for k in range(RPT // C):
        pltpu.sync_copy(rows0, acc.at[pl.ds(sid * RPT + k * C, C)])
    plsc.subcore_barrier()

    def e_load(c, srcb, dstb, ewb, sem):
        off = eb + c * C
        pltpu.async_copy(src_hbm.at[pl.ds(off, C)], srcb, sem)
        pltpu.async_copy(dst_hbm.at[pl.ds(off, C)], dstb, sem)
        pltpu.async_copy(ew_hbm.at[pl.ds(off, C)], ewb, sem)

    def e_wait(c, srcb, dstb, ewb, sem):
        off = eb + c * C
        pltpu.make_async_copy(src_hbm.at[pl.ds(off, C)], srcb, sem).wait()
        pltpu.make_async_copy(dst_hbm.at[pl.ds(off, C)], dstb, sem).wait()
        pltpu.make_async_copy(ew_hbm.at[pl.ds(off, C)], ewb, sem).wait()

    def gather(srcb, rows, sem):
        pltpu.async_copy(hs_hbm.at[srcb], rows, sem)

    def gather_wait(srcb, rows, sem):
        pltpu.make_async_copy(hs_hbm.at[srcb], rows, sem).wait()

    def process(srcb, dstb, ewb, rows, dstS, semS):
        for g in range(_G):
            sl = pl.ds(g * 16, 16)
            d16 = dstb[sl]
            dstS[sl] = d16
            sbuf[sl] = (ewb[sl] * plsc.load_gather(disv, [srcb[sl]])
                        * plsc.load_gather(disv, [d16]))

        def gbody(g, carry):
            sv = sbuf[pl.ds(g * 16, 16)]
            base = g * 16
            for k in range(16):
                s = sv[k]
                for j in range(F // 16):
                    rows[base + k, pl.ds(j * 16, 16)] = (
                        rows[base + k, pl.ds(j * 16, 16)] * s)
            return carry

        lax.fori_loop(0, _G, gbody, 0)
        pltpu.async_copy(rows, acc.at[dstS], semS, add=True)

    def scatter_wait(rows, dstS, semS):
        pltpu.make_async_copy(rows, acc.at[dstS], semS).wait()

    buf0 = (src0, dst0, ew0, rows0, dS0, sem0, semS0)
    buf1 = (src1, dst1, ew1, rows1, dS1, sem1, semS1)
    buf2 = (src2, dst2, ew2, rows2, dS2, sem2, semS2)

    e_load(0, src0, dst0, ew0, sem0)
    e_load(1, src1, dst1, ew1, sem1)
    e_load(2, src2, dst2, ew2, sem2)
    e_wait(0, src0, dst0, ew0, sem0)
    gather(src0, rows0, sem0)
    e_wait(1, src1, dst1, ew1, sem1)
    gather(src1, rows1, sem1)

    def step(i, cur, nxt2):
        srcb, dstb, ewb, rows, dstS, sem, semS = cur
        srcN, dstN, ewN, rowsN, dstSN, semN, semSN = nxt2

        @pl.when(i < NCH - 2)
        def _():
            e_wait(i + 2, srcN, dstN, ewN, semN)

            @pl.when(i >= 1)
            def _():
                scatter_wait(rowsN, dstSN, semSN)

            gather(srcN, rowsN, semN)

        gather_wait(srcb, rows, sem)
        process(srcb, dstb, ewb, rows, dstS, semS)

        @pl.when(i < NCH - 3)
        def _():
            e_load(i + 3, srcb, dstb, ewb, sem)

    def body(i, carry):
        step(3 * i, buf0, buf2)
        step(3 * i + 1, buf1, buf0)
        step(3 * i + 2, buf2, buf1)
        return carry

    lax.fori_loop(0, NCH // 3, body, 0)

    scatter_wait(rows0, dS0, semS0)
    scatter_wait(rows1, dS1, semS1)
    scatter_wait(rows2, dS2, semS2)
    plsc.subcore_barrier()
    for k in range(RPT // C):
        r0 = sid * RPT + k * C
        pltpu.sync_copy(acc.at[pl.ds(r0, C)],
                        out_hbm.at[cid, pl.ds(r0, C)])


def _post_body(accp_ref, b_ref, g_ref, be_ref, rm_ref, rv_ref, lg_ref, lb_ref,
               o_ref):
    o = accp_ref[0] + accp_ref[1] + b_ref[...]
    o = jnp.where(o > 0, o, jnp.exp(o) - 1.0)
    o = (o - rm_ref[...]) * lax.rsqrt(rv_ref[...] + 1e-5) * g_ref[...] + be_ref[...]
    mu = jnp.mean(o, axis=-1, keepdims=True)
    var = jnp.mean((o - mu) ** 2, axis=-1, keepdims=True)
    o_ref[...] = (o - mu) * lax.rsqrt(var + 1e-5) * lg_ref[...] + lb_ref[...]


_vec_spec = pl.BlockSpec((1, F), lambda i: (0, 0))

_tc_post = pl.pallas_call(
    _post_body,
    grid=(NPAD // _RB,),
    in_specs=[pl.BlockSpec((NC, _RB, F), lambda i: (0, i, 0))] + [_vec_spec] * 7,
    out_specs=pl.BlockSpec((_RB, F), lambda i: (i, 0)),
    out_shape=jax.ShapeDtypeStruct((NPAD, F), jnp.float32),
)


def kernel(x, edge_index, edge_weight, W, b, bn_gamma, bn_beta,
           running_mean, running_var, ln_gamma, ln_beta):
    loop = jnp.arange(NPAD, dtype=jnp.int32)
    npadedge = E2 - E - NPAD
    src2 = jnp.concatenate(
        [edge_index[0], loop, jnp.zeros((npadedge,), jnp.int32)])
    dst2 = jnp.concatenate(
        [edge_index[1], loop, jnp.zeros((npadedge,), jnp.int32)])
    ew2 = jnp.concatenate(
        [edge_weight, jnp.ones((NPAD,), jnp.float32),
         jnp.zeros((npadedge,), jnp.float32)])
    xpad = jnp.pad(x, ((0, NPAD - N), (0, 0)))

    degp = _deg_kernel(dst2, ew2)
    hs, dis = _tc_mid(xpad, W, degp)
    accp = _msg_kernel(hs, src2, dst2, ew2, dis)
    out = _tc_post(accp, b.reshape(1, F), bn_gamma.reshape(1, F),
                   bn_beta.reshape(1, F), running_mean.reshape(1, F),
                   running_var.reshape(1, F), ln_gamma.reshape(1, F),
                   ln_beta.reshape(1, F))
    return out[:N]


# speedup vs baseline: 2.5979x; 1.0335x over previous
import functools

import jax
import jax.numpy as jnp
from jax import lax
from jax.experimental import pallas as pl
from jax.experimental.pallas import tpu as pltpu
from jax.experimental.pallas import tpu_sc as plsc

N = 10000
NPAD = 10240
E = 320000
F = 128
NC = 2
NS = 16
NW = NC * NS
C = 80
NCH = 129
EPT = NCH * C
E2 = EPT * NW
RPT = NPAD // NS
ZR = 128

_mesh = plsc.VectorSubcoreMesh(core_axis_name="c", subcore_axis_name="s")


@functools.partial(
    pl.kernel,
    out_type=jax.ShapeDtypeStruct((NC, NPAD), jnp.float32),
    mesh=_mesh,
    scratch_types=[
        pltpu.VMEM_SHARED((NPAD,), jnp.float32),
        pltpu.VMEM((EPT,), jnp.int32),
        pltpu.VMEM((EPT,), jnp.float32),
        pltpu.VMEM((C,), jnp.int32),
        pltpu.VMEM((C,), jnp.float32),
        pltpu.VMEM((RPT,), jnp.float32),
    ],
)
def _deg_kernel(dst_hbm, ew_hbm, out_hbm, acc, dstall, ewall, dstbuf, ewbuf, zb):
    cid = lax.axis_index("c")
    sid = lax.axis_index("s")
    wid = cid * NS + sid
    eb = wid * EPT
    pltpu.sync_copy(dst_hbm.at[pl.ds(eb, EPT)], dstall)
    pltpu.sync_copy(ew_hbm.at[pl.ds(eb, EPT)], ewall)

    def zb_body(i, carry):
        zb[pl.ds(i * 16, 16)] = jnp.zeros((16,), jnp.float32)
        return carry

    lax.fori_loop(0, RPT // 16, zb_body, 0)
    pltpu.sync_copy(zb, acc.at[pl.ds(sid * RPT, RPT)])
    plsc.subcore_barrier()

    def ch_body(c, carry):
        off = c * C
        for g in range(C // 16):
            dstbuf[pl.ds(g * 16, 16)] = dstall[pl.ds(off + g * 16, 16)]
            ewbuf[pl.ds(g * 16, 16)] = ewall[pl.ds(off + g * 16, 16)]
        pltpu.sync_copy(ewbuf, acc.at[dstbuf], add=True)
        return carry

    lax.fori_loop(0, NCH, ch_body, 0)
    plsc.subcore_barrier()
    pltpu.sync_copy(acc.at[pl.ds(sid * RPT, RPT)],
                    out_hbm.at[cid, pl.ds(sid * RPT, RPT)])


def _mm_body(x_ref, w_ref, hs_ref):
    hs_ref[...] = jnp.dot(x_ref[...], w_ref[...],
                          preferred_element_type=jnp.float32)


_RB = 1024

_tc_matmul = pl.pallas_call(
    _mm_body,
    grid=(NPAD // _RB,),
    in_specs=[
        pl.BlockSpec((_RB, F), lambda i: (i, 0)),
        pl.BlockSpec((F, F), lambda i: (0, 0)),
    ],
    out_specs=pl.BlockSpec((_RB, F), lambda i: (i, 0)),
    out_shape=jax.ShapeDtypeStruct((NPAD, F), jnp.float32),
)


def _dis_body(degp_ref, dis_ref):
    deg = degp_ref[0, :] + degp_ref[1, :]
    dis_ref[...] = jnp.where(deg > 0, lax.rsqrt(deg), 0.0)[None, :]


_tc_dis = pl.pallas_call(
    _dis_body,
    grid=(1,),
    in_specs=[pl.BlockSpec((NC, NPAD), lambda i: (0, 0))],
    out_specs=pl.BlockSpec((1, NPAD), lambda i: (0, 0)),
    out_shape=jax.ShapeDtypeStruct((1, NPAD), jnp.float32),
)


_G = C // 16


@functools.partial(
    pl.kernel,
    out_type=jax.ShapeDtypeStruct((NC, NPAD, F), jnp.float32),
    mesh=_mesh,
    scratch_types=[
        pltpu.VMEM_SHARED((NPAD, F), jnp.float32),
        pltpu.VMEM((NPAD,), jnp.float32),
        pltpu.VMEM((C,), jnp.float32),
    ] + [pltpu.VMEM((C,), jnp.int32)] * 3
      + [pltpu.VMEM((C,), jnp.int32)] * 3
      + [pltpu.VMEM((C,), jnp.float32)] * 3
      + [pltpu.VMEM((C, F), jnp.float32)] * 3
      + [pltpu.VMEM((C,), jnp.int32)] * 3
      + [pltpu.SemaphoreType.DMA] * 6,
    compiler_params=pltpu.CompilerParams(needs_layout_passes=False),
)
def _msg_kernel(hs_hbm, src_hbm, dst_hbm, ew_hbm, dis_hbm, out_hbm,
                acc, disv, sbuf, src0, src1, src2, dst0, dst1, dst2,
                ew0, ew1, ew2, rows0, rows1, rows2, dS0, dS1, dS2,
                sem0, sem1, sem2, semS0, semS1, semS2):
    cid = lax.axis_index("c")
    sid = lax.axis_index("s")
    wid = cid * NS + sid
    eb = wid * EPT
    pltpu.sync_copy(dis_hbm.at[0], disv)

    def zb_body(i, carry):
        for j in range(F // 16):
            rows0[i, pl.ds(j * 16, 16)] = jnp.zeros((16,), jnp.float32)
        return carry

    lax.fori_loop(0, C, zb_body, 0)
    for k in range(RPT // C):
        pltpu.sync_copy(rows0, acc.at[pl.ds(sid * RPT + k * C, C)])
    plsc.subcore_barrier()

    def e_load(c, srcb, dstb, ewb, sem):
        off = eb + c * C
        pltpu.async_copy(src_hbm.at[pl.ds(off, C)], srcb, sem)
        pltpu.async_copy(dst_hbm.at[pl.ds(off, C)], dstb, sem)
        pltpu.async_copy(ew_hbm.at[pl.ds(off, C)], ewb, sem)

    def e_wait(c, srcb, dstb, ewb, sem):
        off = eb + c * C
        pltpu.make_async_copy(src_hbm.at[pl.ds(off, C)], srcb, sem).wait()
        pltpu.make_async_copy(dst_hbm.at[pl.ds(off, C)], dstb, sem).wait()
        pltpu.make_async_copy(ew_hbm.at[pl.ds(off, C)], ewb, sem).wait()

    def gather(srcb, rows, sem):
        pltpu.async_copy(hs_hbm.at[srcb], rows, sem)

    def gather_wait(srcb, rows, sem):
        pltpu.make_async_copy(hs_hbm.at[srcb], rows, sem).wait()

    def process(srcb, dstb, ewb, rows, dstS, semS):
        for g in range(_G):
            sl = pl.ds(g * 16, 16)
            d16 = dstb[sl]
            dstS[sl] = d16
            sbuf[sl] = (ewb[sl] * plsc.load_gather(disv, [srcb[sl]])
                        * plsc.load_gather(disv, [d16]))

        def gbody(g, carry):
            sv = sbuf[pl.ds(g * 16, 16)]
            base = g * 16
            for k in range(16):
                s = sv[k]
                for j in range(F // 16):
                    rows[base + k, pl.ds(j * 16, 16)] = (
                        rows[base + k, pl.ds(j * 16, 16)] * s)
            return carry

        lax.fori_loop(0, _G, gbody, 0)
        pltpu.async_copy(rows, acc.at[dstS], semS, add=True)

    def scatter_wait(rows, dstS, semS):
        pltpu.make_async_copy(rows, acc.at[dstS], semS).wait()

    buf0 = (src0, dst0, ew0, rows0, dS0, sem0, semS0)
    buf1 = (src1, dst1, ew1, rows1, dS1, sem1, semS1)
    buf2 = (src2, dst2, ew2, rows2, dS2, sem2, semS2)

    e_load(0, src0, dst0, ew0, sem0)
    e_load(1, src1, dst1, ew1, sem1)
    e_load(2, src2, dst2, ew2, sem2)
    e_wait(0, src0, dst0, ew0, sem0)
    gather(src0, rows0, sem0)
    e_wait(1, src1, dst1, ew1, sem1)
    gather(src1, rows1, sem1)

    def step(i, cur, nxt2):
        srcb, dstb, ewb, rows, dstS, sem, semS = cur
        srcN, dstN, ewN, rowsN, dstSN, semN, semSN = nxt2

        @pl.when(i < NCH - 2)
        def _():
            e_wait(i + 2, srcN, dstN, ewN, semN)

            @pl.when(i >= 1)
            def _():
                scatter_wait(rowsN, dstSN, semSN)

            gather(srcN, rowsN, semN)

        gather_wait(srcb, rows, sem)
        process(srcb, dstb, ewb, rows, dstS, semS)

        @pl.when(i < NCH - 3)
        def _():
            e_load(i + 3, srcb, dstb, ewb, sem)

    def body(i, carry):
        step(3 * i, buf0, buf2)
        step(3 * i + 1, buf1, buf0)
        step(3 * i + 2, buf2, buf1)
        return carry

    lax.fori_loop(0, NCH // 3, body, 0)

    scatter_wait(rows0, dS0, semS0)
    scatter_wait(rows1, dS1, semS1)
    scatter_wait(rows2, dS2, semS2)
    plsc.subcore_barrier()
    for k in range(RPT // C):
        r0 = sid * RPT + k * C
        pltpu.sync_copy(acc.at[pl.ds(r0, C)],
                        out_hbm.at[cid, pl.ds(r0, C)])


def _post_body(accp_ref, b_ref, g_ref, be_ref, rm_ref, rv_ref, lg_ref, lb_ref,
               o_ref):
    o = accp_ref[0] + accp_ref[1] + b_ref[...]
    o = jnp.where(o > 0, o, jnp.exp(o) - 1.0)
    o = (o - rm_ref[...]) * lax.rsqrt(rv_ref[...] + 1e-5) * g_ref[...] + be_ref[...]
    mu = jnp.mean(o, axis=-1, keepdims=True)
    var = jnp.mean((o - mu) ** 2, axis=-1, keepdims=True)
    o_ref[...] = (o - mu) * lax.rsqrt(var + 1e-5) * lg_ref[...] + lb_ref[...]


_vec_spec = pl.BlockSpec((1, F), lambda i: (0, 0))

_tc_post = pl.pallas_call(
    _post_body,
    grid=(NPAD // _RB,),
    in_specs=[pl.BlockSpec((NC, _RB, F), lambda i: (0, i, 0))] + [_vec_spec] * 7,
    out_specs=pl.BlockSpec((_RB, F), lambda i: (i, 0)),
    out_shape=jax.ShapeDtypeStruct((NPAD, F), jnp.float32),
)


def kernel(x, edge_index, edge_weight, W, b, bn_gamma, bn_beta,
           running_mean, running_var, ln_gamma, ln_beta):
    loop = jnp.arange(NPAD, dtype=jnp.int32)
    npadedge = E2 - E - NPAD
    src2 = jnp.concatenate(
        [edge_index[0], loop, jnp.zeros((npadedge,), jnp.int32)])
    dst2 = jnp.concatenate(
        [edge_index[1], loop, jnp.zeros((npadedge,), jnp.int32)])
    ew2 = jnp.concatenate(
        [edge_weight, jnp.ones((NPAD,), jnp.float32),
         jnp.zeros((npadedge,), jnp.float32)])
    xpad = jnp.pad(x, ((0, NPAD - N), (0, 0)))

    degp = _deg_kernel(dst2, ew2)
    hs = _tc_matmul(xpad, W)
    dis = _tc_dis(degp)
    accp = _msg_kernel(hs, src2, dst2, ew2, dis)
    out = _tc_post(accp, b.reshape(1, F), bn_gamma.reshape(1, F),
                   bn_beta.reshape(1, F), running_mean.reshape(1, F),
                   running_var.reshape(1, F), ln_gamma.reshape(1, F),
                   ln_beta.reshape(1, F))
    return out[:N]
